# baseline (speedup 1.0000x reference)
import jax
import jax.numpy as jnp
from jax.experimental import pallas as pl
from jax.experimental.pallas import tpu as pltpu

B, L, D, NH = 4, 1500, 512, 8
HD = D // NH
LP = 1536
EPS = 1e-8
PEPS = 1.1920929e-07


def _nrm(x):
    n = jnp.sqrt(jnp.sum(x * x, -1, keepdims=True))
    return x / jnp.maximum(n, EPS)


def _fused_kernel(h_ref, u_ref, len_ref, sb_ref, w1_ref, b1_ref, w2_ref,
                  b2_ref, wq_ref, wk_ref, g_ref, bta_ref, wb8_ref, wpv_ref,
                  wpo_ref, o_ref):
    h = h_ref[0]

    y = _nrm(h)
    m = jnp.dot(y, w1_ref[...],
                preferred_element_type=jnp.float32) + b1_ref[...]
    m = 0.5 * m * (1.0 + jax.lax.erf(m * 0.7071067811865476))
    m = jnp.dot(m, w2_ref[...],
                preferred_element_type=jnp.float32) + b2_ref[...]
    t = _nrm(m + y)
    qp = jnp.dot(t, wq_ref[...], preferred_element_type=jnp.float32)
    kp = jnp.dot(t, wk_ref[...], preferred_element_type=jnp.float32)
    kp_sh = jnp.concatenate([kp[1:L], kp[0:1]], axis=0)
    c = qp * kp_sh
    cos_col = jnp.sum(c, -1, keepdims=True)
    cos_b = jnp.concatenate([jnp.broadcast_to(cos_col, (L, 128)),
                             jnp.zeros((LP - L, 128), jnp.float32)], axis=0)
    cos = jnp.swapaxes(cos_b, 0, 1)[0:1, :]

    idx = jax.lax.broadcasted_iota(jnp.int32, (1, LP), 1).astype(jnp.float32)
    probs = jnp.clip((1.0 - (cos + sb_ref[0, 0])) * 0.5, 0.0, 1.0)
    probs = jnp.where(idx >= (L - 1), 0.0, probs)
    p = jnp.clip(probs, PEPS, 1.0 - PEPS)
    u = jnp.concatenate([u_ref[0], jnp.full((1, LP - L), 0.5, jnp.float32)],
                        axis=1)
    u = jnp.clip(u, PEPS, 1.0 - PEPS)
    logit = jnp.log(p) - jnp.log1p(-p) + jnp.log(u) - jnp.log1p(-u)
    soft = jax.nn.sigmoid(logit)
    hard = jnp.where(soft > 0.5, 1.0, 0.0)

    lv = len_ref[0]
    valid_len = jnp.minimum(jnp.trunc(lv * (L + 1)) - 1.0, float(L))
    trunc = valid_len < float(L)
    zmask = trunc & (idx >= valid_len)
    omask = trunc & (idx == valid_len)
    hard = jnp.where(omask, 1.0, jnp.where(zmask, 0.0, hard))
    hard = jnp.where(idx >= L, 0.0, hard)
    nb = jnp.sum(hard, axis=1, keepdims=True) == 0.0
    emerg = jnp.minimum(valid_len, float(L - 1))
    hard = jnp.where(nb & (idx == emerg), 1.0, hard)

    inc = hard
    for d in (1, 2, 4, 8, 16, 32, 64, 128, 256, 512, 1024):
        sh = pltpu.roll(inc, d, axis=1)
        inc = inc + jnp.where(idx < d, 0.0, sh)
    seg_row = (inc - hard)[:, :L]
    vl = jnp.trunc(lv * L)

    mu = jnp.mean(h, -1, keepdims=True)
    hv = h - mu
    hn = hv / jnp.sqrt(jnp.mean(hv * hv, -1, keepdims=True) + 1e-5)
    hn = (hn * g_ref[...] + bta_ref[...]).astype(jnp.bfloat16)
    vals = jnp.dot(hn, wpv_ref[...], preferred_element_type=jnp.float32)
    base8 = jnp.dot(hn, wb8_ref[...], preferred_element_type=jnp.float32)
    gmax8 = jnp.max(base8, axis=0, keepdims=True)
    e8 = jnp.exp(base8 - gmax8)
    col = jax.lax.broadcasted_iota(jnp.int32, (L, 1), 0).astype(jnp.float32)
    e8 = jnp.where(col < vl, e8, 0.0).astype(jnp.bfloat16)
    hi8 = jax.lax.broadcasted_iota(jnp.int32, (128, D), 0)
    hj8 = jax.lax.broadcasted_iota(jnp.int32, (128, D), 1) // HD
    bd8 = jnp.where(hi8 == hj8, 1.0, 0.0)
    bd8b = bd8.astype(jnp.bfloat16)
    ew = jnp.dot(e8, bd8b, preferred_element_type=jnp.float32)
    ev = (ew * vals).astype(jnp.bfloat16)

    sidx = jax.lax.broadcasted_iota(jnp.int32, (128, L), 0).astype(jnp.float32)
    for i in range(LP // 128):
        m2 = jnp.where(sidx == (seg_row - float(128 * i)),
                       1.0, 0.0).astype(jnp.bfloat16)
        num = jnp.dot(m2, ev, preferred_element_type=jnp.float32)
        den8 = jnp.dot(m2, e8, preferred_element_type=jnp.float32)
        den = jnp.dot(den8, bd8, preferred_element_type=jnp.float32)
        pooled = jnp.where(den > 0.0, num / jnp.where(den > 0.0, den, 1.0),
                           0.0).astype(jnp.bfloat16)
        out = jnp.dot(pooled, wpo_ref[...], preferred_element_type=jnp.float32)
        if 128 * (i + 1) <= L:
            o_ref[0, 128 * i:128 * (i + 1), :] = out
        else:
            o_ref[0, 128 * i:L, :] = out[:L - 128 * i, :]


@jax.jit
def kernel(hidden, lengths, u_noise, W1, b1, W2, b2, Wq, Wk, sim_bias,
           learned_query, Wpk, Wpv, Wpo, ln_g, ln_b):
    f32 = jnp.float32
    up = u_noise.reshape(B, 1, L)
    lenr = lengths.reshape(B, 1, 1).astype(f32)
    sb = jnp.asarray(sim_bias, f32).reshape(1, 1)
    lqw = Wpk.T * learned_query[None, :]
    wb8 = jnp.sum(lqw.reshape(D, NH, HD), axis=-1) * (HD ** -0.5)
    wb8 = jnp.pad(wb8, ((0, 0), (0, 128 - NH))).astype(jnp.bfloat16)

    wspec = pl.BlockSpec((D, D), lambda b: (0, 0))
    vspec = pl.BlockSpec((1, D), lambda b: (0, 0))
    hspec = pl.BlockSpec((1, L, D), lambda b: (b, 0, 0))
    uspec = pl.BlockSpec((1, 1, L), lambda b: (b, 0, 0))
    sspec = pl.BlockSpec((1, 1, 1), lambda b: (b, 0, 0))

    out = pl.pallas_call(
        _fused_kernel,
        grid=(B,),
        in_specs=[hspec, uspec, sspec, pl.BlockSpec((1, 1), lambda b: (0, 0)),
                  wspec, vspec, wspec, vspec, wspec, wspec,
                  vspec, vspec, pl.BlockSpec((D, 128), lambda b: (0, 0)),
                  wspec, wspec],
        out_specs=hspec,
        out_shape=jax.ShapeDtypeStruct((B, L, D), f32),
        compiler_params=pltpu.CompilerParams(
            dimension_semantics=("parallel",),
            vmem_limit_bytes=56 * 1024 * 1024,
        ),
        name="boundary_pool_fused",
    )(hidden, up, lenr, sb, W1.T, b1.reshape(1, D), W2.T, b2.reshape(1, D),
      Wq.T, Wk.T, ln_g.reshape(1, D), ln_b.reshape(1, D), wb8,
      Wpv.T.astype(jnp.bfloat16), Wpo.T.astype(jnp.bfloat16))
    return out


# speedup vs baseline: 1.0127x; 1.0127x over previous
import jax
import jax.numpy as jnp
from jax.experimental import pallas as pl
from jax.experimental.pallas import tpu as pltpu

B, L, D, NH = 4, 1500, 512, 8
HD = D // NH
LP = 1536
EPS = 1e-8
PEPS = 1.1920929e-07


def _nrm(x):
    n = jnp.sqrt(jnp.sum(x * x, -1, keepdims=True))
    return x / jnp.maximum(n, EPS)


def _fused_kernel(h_ref, u_ref, len_ref, sb_ref, w1_ref, b1_ref, w2_ref,
                  b2_ref, wq_ref, wk_ref, g_ref, bta_ref, wb8_ref, wpv_ref,
                  wpo_ref, o_ref):
    h = h_ref[0]

    y = _nrm(h)
    m = jnp.dot(y, w1_ref[...],
                preferred_element_type=jnp.float32) + b1_ref[...]
    m = 0.5 * m * (1.0 + jax.lax.erf(m * 0.7071067811865476))
    m = jnp.dot(m, w2_ref[...],
                preferred_element_type=jnp.float32) + b2_ref[...]
    t = _nrm(m + y)
    qp = jnp.dot(t, wq_ref[...], preferred_element_type=jnp.float32)
    kp = jnp.dot(t, wk_ref[...], preferred_element_type=jnp.float32)
    kp_sh = jnp.concatenate([kp[1:L], kp[0:1]], axis=0)
    c = qp * kp_sh
    cos_col = jnp.sum(c, -1, keepdims=True)
    cos_b = jnp.concatenate([jnp.broadcast_to(cos_col, (L, 128)),
                             jnp.zeros((LP - L, 128), jnp.float32)], axis=0)
    cos = jnp.swapaxes(cos_b, 0, 1)[0:1, :]

    idx = jax.lax.broadcasted_iota(jnp.int32, (1, LP), 1).astype(jnp.float32)
    probs = jnp.clip((1.0 - (cos + sb_ref[0, 0])) * 0.5, 0.0, 1.0)
    probs = jnp.where(idx >= (L - 1), 0.0, probs)
    p = jnp.clip(probs, PEPS, 1.0 - PEPS)
    u = jnp.concatenate([u_ref[0], jnp.full((1, LP - L), 0.5, jnp.float32)],
                        axis=1)
    u = jnp.clip(u, PEPS, 1.0 - PEPS)
    logit = jnp.log(p) - jnp.log1p(-p) + jnp.log(u) - jnp.log1p(-u)
    soft = jax.nn.sigmoid(logit)
    hard = jnp.where(soft > 0.5, 1.0, 0.0)

    lv = len_ref[0]
    valid_len = jnp.minimum(jnp.trunc(lv * (L + 1)) - 1.0, float(L))
    trunc = valid_len < float(L)
    zmask = trunc & (idx >= valid_len)
    omask = trunc & (idx == valid_len)
    hard = jnp.where(omask, 1.0, jnp.where(zmask, 0.0, hard))
    hard = jnp.where(idx >= L, 0.0, hard)
    nb = jnp.sum(hard, axis=1, keepdims=True) == 0.0
    emerg = jnp.minimum(valid_len, float(L - 1))
    hard = jnp.where(nb & (idx == emerg), 1.0, hard)

    inc = hard
    for d in (1, 2, 4, 8, 16, 32, 64, 128, 256, 512, 1024):
        sh = pltpu.roll(inc, d, axis=1)
        inc = inc + jnp.where(idx < d, 0.0, sh)
    seg_row = (inc - hard)[:, :L]
    vl = jnp.trunc(lv * L)

    mu = jnp.mean(h, -1, keepdims=True)
    hv = h - mu
    hn = hv / jnp.sqrt(jnp.mean(hv * hv, -1, keepdims=True) + 1e-5)
    hn = hn * g_ref[...] + bta_ref[...]
    vals = jnp.dot(hn, wpv_ref[...], preferred_element_type=jnp.float32)
    base8 = jnp.dot(hn, wb8_ref[...], preferred_element_type=jnp.float32)
    gmax8 = jnp.max(base8, axis=0, keepdims=True)
    e8 = jnp.exp(base8 - gmax8)
    col = jax.lax.broadcasted_iota(jnp.int32, (L, 1), 0).astype(jnp.float32)
    e8 = jnp.where(col < vl, e8, 0.0)
    hi8 = jax.lax.broadcasted_iota(jnp.int32, (128, D), 0)
    hj8 = jax.lax.broadcasted_iota(jnp.int32, (128, D), 1) // HD
    bd8 = jnp.where(hi8 == hj8, 1.0, 0.0)
    ew = jnp.dot(e8, bd8, preferred_element_type=jnp.float32)
    ev = ew * vals

    sidx = jax.lax.broadcasted_iota(jnp.int32, (128, L), 0).astype(jnp.float32)
    for i in range(LP // 128):
        m2 = jnp.where(sidx == (seg_row - float(128 * i)), 1.0, 0.0)
        num = jnp.dot(m2, ev, preferred_element_type=jnp.float32)
        den8 = jnp.dot(m2, e8, preferred_element_type=jnp.float32)
        den = jnp.dot(den8, bd8, preferred_element_type=jnp.float32)
        pooled = jnp.where(den > 0.0, num / jnp.where(den > 0.0, den, 1.0),
                           0.0)
        out = jnp.dot(pooled, wpo_ref[...], preferred_element_type=jnp.float32)
        if 128 * (i + 1) <= L:
            o_ref[0, 128 * i:128 * (i + 1), :] = out
        else:
            o_ref[0, 128 * i:L, :] = out[:L - 128 * i, :]


@jax.jit
def kernel(hidden, lengths, u_noise, W1, b1, W2, b2, Wq, Wk, sim_bias,
           learned_query, Wpk, Wpv, Wpo, ln_g, ln_b):
    f32 = jnp.float32
    up = u_noise.reshape(B, 1, L)
    lenr = lengths.reshape(B, 1, 1).astype(f32)
    sb = jnp.asarray(sim_bias, f32).reshape(1, 1)
    lqw = Wpk.T * learned_query[None, :]
    wb8 = jnp.sum(lqw.reshape(D, NH, HD), axis=-1) * (HD ** -0.5)
    wb8 = jnp.pad(wb8, ((0, 0), (0, 128 - NH)))

    wspec = pl.BlockSpec((D, D), lambda b: (0, 0))
    vspec = pl.BlockSpec((1, D), lambda b: (0, 0))
    hspec = pl.BlockSpec((1, L, D), lambda b: (b, 0, 0))
    uspec = pl.BlockSpec((1, 1, L), lambda b: (b, 0, 0))
    sspec = pl.BlockSpec((1, 1, 1), lambda b: (b, 0, 0))

    out = pl.pallas_call(
        _fused_kernel,
        grid=(B,),
        in_specs=[hspec, uspec, sspec, pl.BlockSpec((1, 1), lambda b: (0, 0)),
                  wspec, vspec, wspec, vspec, wspec, wspec,
                  vspec, vspec, pl.BlockSpec((D, 128), lambda b: (0, 0)),
                  wspec, wspec],
        out_specs=hspec,
        out_shape=jax.ShapeDtypeStruct((B, L, D), f32),
        compiler_params=pltpu.CompilerParams(
            dimension_semantics=("parallel",),
            vmem_limit_bytes=56 * 1024 * 1024,
        ),
        name="boundary_pool_fused",
    )(hidden, up, lenr, sb, W1.T, b1.reshape(1, D), W2.T, b2.reshape(1, D),
      Wq.T, Wk.T, ln_g.reshape(1, D), ln_b.reshape(1, D), wb8,
      Wpv.T, Wpo.T)
    return out
